# baseline (speedup 1.0000x reference)
import functools
import jax
import jax.numpy as jnp
import numpy as np
from jax import lax
from jax.experimental import pallas as pl
from jax.experimental.pallas import tpu as pltpu
from jax.experimental.pallas import tpu_sc as plsc

N = 10000
E = 320000
NFEAT = 128
HIDDEN = 128
CH = 4
PCD = 32

NTILES = 32
FPT = HIDDEN // NTILES
UOFF = HIDDEN * N
VOFF = (HIDDEN + 4) * N

EPT1 = E // NTILES
CH1 = 2000
CH2 = 4000

_mesh = plsc.VectorSubcoreMesh(core_axis_name="c", subcore_axis_name="s")
_sc_params = pltpu.CompilerParams(needs_layout_passes=False)


def _pre_body(w_ref, x_ref, b_ref, o_ref):
    acc = lax.dot_general(
        w_ref[...], x_ref[...], (((1,), (1,)), ((), ())),
        preferred_element_type=jnp.float32,
    )
    o_ref[...] = acc + jnp.broadcast_to(b_ref[...], acc.shape)


def _pre_call(WpT, x, bp):
    return pl.pallas_call(
        _pre_body,
        out_shape=jax.ShapeDtypeStruct((136, N), jnp.float32),
    )(WpT, x, bp)


@functools.partial(
    pl.kernel,
    out_type=jax.ShapeDtypeStruct((CH * E,), jnp.float32),
    mesh=_mesh,
    compiler_params=_sc_params,
    scratch_types=[
        pltpu.VMEM((8 * N,), jnp.float32),
        pltpu.VMEM((CH1,), jnp.int32),
        pltpu.VMEM((CH1,), jnp.int32),
        pltpu.VMEM((CH1,), jnp.int32),
        pltpu.VMEM((CH1,), jnp.int32),
        pltpu.VMEM((4 * CH1,), jnp.float32),
        pltpu.VMEM((4 * CH1,), jnp.float32),
        pltpu.VMEM((16,), jnp.float32),
        pltpu.VMEM((16,), jnp.float32),
        pltpu.SemaphoreType.DMA,
        pltpu.SemaphoreType.DMA,
        pltpu.SemaphoreType.DMA,
        pltpu.SemaphoreType.DMA,
    ],
)
def _sc_alpha(ot_hbm, ei_hbm, a2w_hbm, a2b_hbm, alT_hbm,
              uv_v, col0_v, col1_v, row0_v, row1_v, a0_v, a1_v, w_v, b_v,
              isem0, isem1, osem0, osem1):
    wid = lax.axis_index("s") * 2 + lax.axis_index("c")
    e0 = wid * EPT1
    nchunks = EPT1 // CH1
    cols_v = [col0_v, col1_v]
    rows_v = [row0_v, row1_v]
    as_v = [a0_v, a1_v]
    isems = [isem0, isem1]
    osems = [osem0, osem1]

    def fetch(ci, b):
        base = e0 + ci * CH1
        pltpu.async_copy(ei_hbm.at[pl.ds(E + base, CH1)], cols_v[b], isems[b])
        pltpu.async_copy(ei_hbm.at[pl.ds(base, CH1)], rows_v[b], isems[b])

    def drain_in(b):
        pltpu.make_async_copy(ei_hbm.at[pl.ds(0, CH1)], cols_v[b], isems[b]).wait()
        pltpu.make_async_copy(ei_hbm.at[pl.ds(0, CH1)], rows_v[b], isems[b]).wait()

    def store_out(ci, b):
        base = e0 + ci * CH1
        for c in range(4):
            pltpu.async_copy(
                as_v[b].at[pl.ds(c * CH1, CH1)],
                alT_hbm.at[pl.ds(c * E + base, CH1)],
                osems[b],
            )

    def drain_out(b):
        for c in range(4):
            pltpu.make_async_copy(
                as_v[b].at[pl.ds(c * CH1, CH1)],
                alT_hbm.at[pl.ds(0, CH1)],
                osems[b],
            ).wait()

    fetch(0, 0)
    pltpu.sync_copy(ot_hbm.at[pl.ds(UOFF, 8 * N)], uv_v)
    pltpu.sync_copy(a2w_hbm, w_v)
    pltpu.sync_copy(a2b_hbm, b_v)
    wvec = w_v[...]
    bvec = b_v[...]
    ws = [[wvec[c * 4 + j] for j in range(4)] for c in range(4)]
    bs = [bvec[c] for c in range(4)]

    for ci in range(nchunks):
        b = ci & 1
        if ci + 1 < nchunks:
            fetch(ci + 1, 1 - b)
        drain_in(b)
        if ci >= 2:
            drain_out(b)

        def g_body(g, _):
            off = g * 16
            cols = cols_v[b][pl.ds(off, 16)]
            rows = rows_v[b][pl.ds(off, 16)] + (4 * N)
            t = [
                plsc.load_gather(uv_v, [cols + (c * N)])
                + plsc.load_gather(uv_v, [rows + (c * N)])
                for c in range(4)
            ]
            s = [
                bs[c]
                + t[0] * ws[c][0]
                + t[1] * ws[c][1]
                + t[2] * ws[c][2]
                + t[3] * ws[c][3]
                for c in range(4)
            ]
            m = jnp.maximum(jnp.maximum(s[0], s[1]), jnp.maximum(s[2], s[3]))
            ex = [jnp.exp(s[c] - m) for c in range(4)]
            r = 1.0 / ((ex[0] + ex[1]) + (ex[2] + ex[3]))
            for c in range(4):
                as_v[b][pl.ds(c * CH1 + off, 16)] = ex[c] * r
            return 0

        lax.fori_loop(0, CH1 // 16, g_body, 0, unroll=2)
        store_out(ci, b)

    drain_out((nchunks - 2) & 1)
    drain_out((nchunks - 1) & 1)


@functools.partial(
    pl.kernel,
    out_type=jax.ShapeDtypeStruct((HIDDEN * N,), jnp.float32),
    mesh=_mesh,
    compiler_params=_sc_params,
    scratch_types=[
        pltpu.VMEM((FPT * N,), jnp.float32),
        pltpu.VMEM((FPT * N,), jnp.float32),
        pltpu.VMEM((CH2,), jnp.int32),
        pltpu.VMEM((CH2,), jnp.int32),
        pltpu.VMEM((CH2,), jnp.int32),
        pltpu.VMEM((CH2,), jnp.int32),
        pltpu.VMEM((CH2,), jnp.float32),
        pltpu.VMEM((CH2,), jnp.float32),
        pltpu.SemaphoreType.DMA,
        pltpu.SemaphoreType.DMA,
    ],
)
def _sc_agg(ot_hbm, ei_hbm, alT_hbm, pout_hbm,
            ct_v, out_v, col0_v, col1_v, row0_v, row1_v, al0_v, al1_v,
            sem0, sem1):
    wid = lax.axis_index("s") * 2 + lax.axis_index("c")
    ch = wid // (NTILES // CH)
    sems = [sem0, sem1]
    cols_v = [col0_v, col1_v]
    rows_v = [row0_v, row1_v]
    als_v = [al0_v, al1_v]
    nchunks = E // CH2

    def start_fetch(ci, b):
        base = ci * CH2
        pltpu.async_copy(ei_hbm.at[pl.ds(E + base, CH2)], cols_v[b], sems[b])
        pltpu.async_copy(ei_hbm.at[pl.ds(base, CH2)], rows_v[b], sems[b])
        pltpu.async_copy(alT_hbm.at[pl.ds(ch * E + base, CH2)], als_v[b], sems[b])

    def drain(b):
        pltpu.make_async_copy(ei_hbm.at[pl.ds(0, CH2)], cols_v[b], sems[b]).wait()
        pltpu.make_async_copy(ei_hbm.at[pl.ds(0, CH2)], rows_v[b], sems[b]).wait()
        pltpu.make_async_copy(alT_hbm.at[pl.ds(0, CH2)], als_v[b], sems[b]).wait()

    start_fetch(0, 0)
    pltpu.sync_copy(ot_hbm.at[pl.ds(wid * (FPT * N), FPT * N)], ct_v)

    def zero_body(i, _):
        out_v[pl.ds(i * 16, 16)] = jnp.zeros((16,), jnp.float32)
        return 0

    lax.fori_loop(0, FPT * N // 16, zero_body, 0, unroll=10)

    def process(b):
        def g_body(g, _):
            off = g * 16
            cols = cols_v[b][pl.ds(off, 16)]
            rows = rows_v[b][pl.ds(off, 16)]
            a = als_v[b][pl.ds(off, 16)]
            for f in range(FPT):
                vals = plsc.load_gather(ct_v, [cols + (f * N)]) * a
                plsc.addupdate_scatter(out_v, [rows + (f * N)], vals)
            return 0

        lax.fori_loop(0, CH2 // 16, g_body, 0)

    def pair_body(ci2, _):
        for b in range(2):
            ci = ci2 * 2 + b
            nxt = jnp.minimum(ci + 1, nchunks - 1)
            start_fetch(nxt, 1 - b)
            drain(b)
            process(b)
        return 0

    lax.fori_loop(0, nchunks // 2, pair_body, 0)
    drain(0)
    pltpu.sync_copy(out_v, pout_hbm.at[pl.ds(wid * (FPT * N), FPT * N)])


def _post_body(st_ref, cb_ref, bm_ref, wc_ref, bc_ref, h_ref, o_ref):
    y = st_ref[...] + jnp.broadcast_to(cb_ref[...], st_ref.shape)
    nsq = jnp.dot(bm_ref[...], y * y, preferred_element_type=jnp.float32)
    ht = y / jnp.maximum(jnp.sqrt(nsq), 1e-12)
    h = ht.T
    h_ref[...] = h
    o_ref[...] = (
        jnp.dot(h, wc_ref[...], preferred_element_type=jnp.float32) + bc_ref[...]
    )[:, :2]


def _post_call(ST, cb, Bmask, Wc, bc):
    return pl.pallas_call(
        _post_body,
        out_shape=[
            jax.ShapeDtypeStruct((N, 128), jnp.float32),
            jax.ShapeDtypeStruct((N, 2), jnp.float32),
        ],
    )(ST, cb, Bmask, Wc, bc)


_BMASK_NP = np.kron(np.eye(CH, dtype=np.float32), np.ones((PCD, PCD), np.float32))


def kernel(x, edge_index, A1_W, A1_b, A2_W, A2_b, lin_W, lin_b, conv_W,
           ch_bias, cls_W, cls_b):
    ei_flat = edge_index.astype(jnp.int32).reshape(-1)

    M = jnp.einsum("kij,kjf->kif", conv_W, lin_W).reshape(HIDDEN, NFEAT)
    bc = jnp.einsum("kij,kj->ki", conv_W, lin_b).reshape(HIDDEN)
    A1a = A1_W[:, :NFEAT]
    A1b = A1_W[:, NFEAT:]
    WpT = jnp.concatenate([M, A1a, A1b], axis=0)
    bp = jnp.concatenate([bc, jnp.zeros((4,), jnp.float32), A1_b])

    OT = _pre_call(WpT, x, bp[:, None])
    ot_flat = OT.reshape(-1)

    a2w = A2_W.reshape(-1)
    a2b = jnp.concatenate([A2_b, jnp.zeros((12,), jnp.float32)])
    alT = _sc_alpha(ot_flat, ei_flat, a2w, a2b)

    pout = _sc_agg(ot_flat, ei_flat, alT)
    ST = pout.reshape(HIDDEN, N)

    Wc = jnp.zeros((HIDDEN, 128), jnp.float32).at[:, :2].set(cls_W.T)
    bcl = jnp.zeros((128,), jnp.float32).at[:2].set(cls_b)
    bmask = jnp.asarray(_BMASK_NP)
    h, out = _post_call(ST, ch_bias.reshape(HIDDEN, 1), bmask, Wc, bcl[None, :])
    return (h, out)


# speedup vs baseline: 1.0071x; 1.0071x over previous
import functools
import jax
import jax.numpy as jnp
from jax import lax
from jax.experimental import pallas as pl
from jax.experimental.pallas import tpu as pltpu
from jax.experimental.pallas import tpu_sc as plsc

N = 10000
E = 320000
NFEAT = 128
HIDDEN = 128
CH = 4
PCD = 32

NTILES = 32
FPT = HIDDEN // NTILES
UOFF = HIDDEN * N
VOFF = (HIDDEN + 4) * N

EPT1 = E // NTILES
CH1 = 2000
CH2 = 4000

_mesh = plsc.VectorSubcoreMesh(core_axis_name="c", subcore_axis_name="s")
_sc_params = pltpu.CompilerParams(needs_layout_passes=False)


def _pre_body(conv_ref, lin_ref, linb_ref, a1_ref, a1b_ref, x_ref, o_ref):
    mrows = [
        jnp.dot(conv_ref[k], lin_ref[k], preferred_element_type=jnp.float32)
        for k in range(CH)
    ]
    w = jnp.concatenate(mrows + [a1_ref[:, :NFEAT], a1_ref[:, NFEAT:]], axis=0)
    acc = lax.dot_general(
        w, x_ref[...], (((1,), (1,)), ((), ())),
        preferred_element_type=jnp.float32,
    )
    bcs = [
        jnp.dot(conv_ref[k], linb_ref[...][k, :, None],
                preferred_element_type=jnp.float32)
        for k in range(CH)
    ]
    bvec = jnp.concatenate(
        bcs + [jnp.zeros((4, 1), jnp.float32), a1b_ref[...].reshape(4, 1)],
        axis=0,
    )
    o_ref[...] = acc + jnp.broadcast_to(bvec, acc.shape)


def _pre_call(conv_W, lin_W, lin_b, A1_W, A1_b, x):
    return pl.pallas_call(
        _pre_body,
        out_shape=jax.ShapeDtypeStruct((136, N), jnp.float32),
    )(conv_W, lin_W, lin_b, A1_W, A1_b, x)


@functools.partial(
    pl.kernel,
    out_type=jax.ShapeDtypeStruct((CH * E,), jnp.float32),
    mesh=_mesh,
    compiler_params=_sc_params,
    scratch_types=[
        pltpu.VMEM((8 * N,), jnp.float32),
        pltpu.VMEM((CH1,), jnp.int32),
        pltpu.VMEM((CH1,), jnp.int32),
        pltpu.VMEM((CH1,), jnp.int32),
        pltpu.VMEM((CH1,), jnp.int32),
        pltpu.VMEM((4 * CH1,), jnp.float32),
        pltpu.VMEM((4 * CH1,), jnp.float32),
        pltpu.VMEM((16,), jnp.float32),
        pltpu.VMEM((16,), jnp.float32),
        pltpu.SemaphoreType.DMA,
        pltpu.SemaphoreType.DMA,
        pltpu.SemaphoreType.DMA,
        pltpu.SemaphoreType.DMA,
    ],
)
def _sc_alpha(ot_hbm, ei_hbm, a2w_hbm, a2b_hbm, alT_hbm,
              uv_v, col0_v, col1_v, row0_v, row1_v, a0_v, a1_v, w_v, b_v,
              isem0, isem1, osem0, osem1):
    wid = lax.axis_index("s") * 2 + lax.axis_index("c")
    e0 = wid * EPT1
    nchunks = EPT1 // CH1
    cols_v = [col0_v, col1_v]
    rows_v = [row0_v, row1_v]
    as_v = [a0_v, a1_v]
    isems = [isem0, isem1]
    osems = [osem0, osem1]

    def fetch(ci, b):
        base = e0 + ci * CH1
        pltpu.async_copy(ei_hbm.at[pl.ds(E + base, CH1)], cols_v[b], isems[b])
        pltpu.async_copy(ei_hbm.at[pl.ds(base, CH1)], rows_v[b], isems[b])

    def drain_in(b):
        pltpu.make_async_copy(ei_hbm.at[pl.ds(0, CH1)], cols_v[b], isems[b]).wait()
        pltpu.make_async_copy(ei_hbm.at[pl.ds(0, CH1)], rows_v[b], isems[b]).wait()

    def store_out(ci, b):
        base = e0 + ci * CH1
        for c in range(4):
            pltpu.async_copy(
                as_v[b].at[pl.ds(c * CH1, CH1)],
                alT_hbm.at[pl.ds(c * E + base, CH1)],
                osems[b],
            )

    def drain_out(b):
        for c in range(4):
            pltpu.make_async_copy(
                as_v[b].at[pl.ds(c * CH1, CH1)],
                alT_hbm.at[pl.ds(0, CH1)],
                osems[b],
            ).wait()

    fetch(0, 0)
    pltpu.sync_copy(ot_hbm.at[pl.ds(UOFF, 8 * N)], uv_v)
    pltpu.sync_copy(a2w_hbm, w_v)
    pltpu.sync_copy(a2b_hbm, b_v)
    wvec = w_v[...]
    bvec = b_v[...]
    ws = [[wvec[c * 4 + j] for j in range(4)] for c in range(4)]
    bs = [bvec[c] for c in range(4)]

    for ci in range(nchunks):
        b = ci & 1
        if ci + 1 < nchunks:
            fetch(ci + 1, 1 - b)
        drain_in(b)
        if ci >= 2:
            drain_out(b)

        def g_body(g, _):
            off = g * 16
            cols = cols_v[b][pl.ds(off, 16)]
            rows = rows_v[b][pl.ds(off, 16)] + (4 * N)
            t = [
                plsc.load_gather(uv_v, [cols + (c * N)])
                + plsc.load_gather(uv_v, [rows + (c * N)])
                for c in range(4)
            ]
            s = [
                bs[c]
                + t[0] * ws[c][0]
                + t[1] * ws[c][1]
                + t[2] * ws[c][2]
                + t[3] * ws[c][3]
                for c in range(4)
            ]
            m = jnp.maximum(jnp.maximum(s[0], s[1]), jnp.maximum(s[2], s[3]))
            ex = [jnp.exp(s[c] - m) for c in range(4)]
            r = 1.0 / ((ex[0] + ex[1]) + (ex[2] + ex[3]))
            for c in range(4):
                as_v[b][pl.ds(c * CH1 + off, 16)] = ex[c] * r
            return 0

        lax.fori_loop(0, CH1 // 16, g_body, 0, unroll=2)
        store_out(ci, b)

    drain_out((nchunks - 2) & 1)
    drain_out((nchunks - 1) & 1)


@functools.partial(
    pl.kernel,
    out_type=jax.ShapeDtypeStruct((HIDDEN * N,), jnp.float32),
    mesh=_mesh,
    compiler_params=_sc_params,
    scratch_types=[
        pltpu.VMEM((FPT * N,), jnp.float32),
        pltpu.VMEM((FPT * N,), jnp.float32),
        pltpu.VMEM((CH2,), jnp.int32),
        pltpu.VMEM((CH2,), jnp.int32),
        pltpu.VMEM((CH2,), jnp.int32),
        pltpu.VMEM((CH2,), jnp.int32),
        pltpu.VMEM((CH2,), jnp.float32),
        pltpu.VMEM((CH2,), jnp.float32),
        pltpu.SemaphoreType.DMA,
        pltpu.SemaphoreType.DMA,
    ],
)
def _sc_agg(ot_hbm, ei_hbm, alT_hbm, pout_hbm,
            ct_v, out_v, col0_v, col1_v, row0_v, row1_v, al0_v, al1_v,
            sem0, sem1):
    wid = lax.axis_index("s") * 2 + lax.axis_index("c")
    ch = wid // (NTILES // CH)
    sems = [sem0, sem1]
    cols_v = [col0_v, col1_v]
    rows_v = [row0_v, row1_v]
    als_v = [al0_v, al1_v]
    nchunks = E // CH2

    def start_fetch(ci, b):
        base = ci * CH2
        pltpu.async_copy(ei_hbm.at[pl.ds(E + base, CH2)], cols_v[b], sems[b])
        pltpu.async_copy(ei_hbm.at[pl.ds(base, CH2)], rows_v[b], sems[b])
        pltpu.async_copy(alT_hbm.at[pl.ds(ch * E + base, CH2)], als_v[b], sems[b])

    def drain(b):
        pltpu.make_async_copy(ei_hbm.at[pl.ds(0, CH2)], cols_v[b], sems[b]).wait()
        pltpu.make_async_copy(ei_hbm.at[pl.ds(0, CH2)], rows_v[b], sems[b]).wait()
        pltpu.make_async_copy(alT_hbm.at[pl.ds(0, CH2)], als_v[b], sems[b]).wait()

    start_fetch(0, 0)
    pltpu.sync_copy(ot_hbm.at[pl.ds(wid * (FPT * N), FPT * N)], ct_v)

    def zero_body(i, _):
        out_v[pl.ds(i * 16, 16)] = jnp.zeros((16,), jnp.float32)
        return 0

    lax.fori_loop(0, FPT * N // 16, zero_body, 0, unroll=10)

    def process(b):
        def g_body(g, _):
            off = g * 16
            cols = cols_v[b][pl.ds(off, 16)]
            rows = rows_v[b][pl.ds(off, 16)]
            a = als_v[b][pl.ds(off, 16)]
            for f in range(FPT):
                vals = plsc.load_gather(ct_v, [cols + (f * N)]) * a
                plsc.addupdate_scatter(out_v, [rows + (f * N)], vals)
            return 0

        lax.fori_loop(0, CH2 // 16, g_body, 0)

    def pair_body(ci2, _):
        for b in range(2):
            ci = ci2 * 2 + b
            nxt = jnp.minimum(ci + 1, nchunks - 1)
            start_fetch(nxt, 1 - b)
            drain(b)
            process(b)
        return 0

    lax.fori_loop(0, nchunks // 2, pair_body, 0)
    drain(0)
    pltpu.sync_copy(out_v, pout_hbm.at[pl.ds(wid * (FPT * N), FPT * N)])


def _post_body(st_ref, cb_ref, wc_ref, bc_ref, h_ref, o_ref):
    y = st_ref[...] + jnp.broadcast_to(cb_ref[...], st_ref.shape)
    ri = lax.broadcasted_iota(jnp.int32, (HIDDEN, HIDDEN), 0) // PCD
    ci = lax.broadcasted_iota(jnp.int32, (HIDDEN, HIDDEN), 1) // PCD
    bm = jnp.where(ri == ci, 1.0, 0.0).astype(jnp.float32)
    nsq = jnp.dot(bm, y * y, preferred_element_type=jnp.float32)
    ht = y / jnp.maximum(jnp.sqrt(nsq), 1e-12)
    h = ht.T
    h_ref[...] = h
    o_ref[...] = (
        lax.dot_general(h, wc_ref[...], (((1,), (1,)), ((), ())),
                        preferred_element_type=jnp.float32)
        + jnp.broadcast_to(bc_ref[...], (N, 2))
    )


def _post_call(ST, ch_bias, cls_W, cls_b):
    return pl.pallas_call(
        _post_body,
        out_shape=[
            jax.ShapeDtypeStruct((N, 128), jnp.float32),
            jax.ShapeDtypeStruct((N, 2), jnp.float32),
        ],
    )(ST, ch_bias, cls_W, cls_b)


def kernel(x, edge_index, A1_W, A1_b, A2_W, A2_b, lin_W, lin_b, conv_W,
           ch_bias, cls_W, cls_b):
    ei_flat = edge_index.astype(jnp.int32).reshape(-1)

    OT = _pre_call(conv_W, lin_W, lin_b, A1_W, A1_b, x)
    ot_flat = OT.reshape(-1)

    a2w = A2_W.reshape(-1)
    a2b = jnp.concatenate([A2_b, jnp.zeros((12,), jnp.float32)])
    alT = _sc_alpha(ot_flat, ei_flat, a2w, a2b)

    pout = _sc_agg(ot_flat, ei_flat, alT)
    ST = pout.reshape(HIDDEN, N)

    h, out = _post_call(ST, ch_bias.reshape(HIDDEN, 1), cls_W, cls_b[None, :])
    return (h, out)


# speedup vs baseline: 1.0429x; 1.0356x over previous
import functools
import jax
import jax.numpy as jnp
from jax import lax
from jax.experimental import pallas as pl
from jax.experimental.pallas import tpu as pltpu
from jax.experimental.pallas import tpu_sc as plsc

N = 10000
E = 320000
NFEAT = 128
HIDDEN = 128
CH = 4
PCD = 32

NTILES = 32
FPT = HIDDEN // NTILES
UOFF = HIDDEN * N
VOFF = (HIDDEN + 4) * N

EPT1 = E // NTILES
CH1 = 2000
CH2 = 4000

_mesh = plsc.VectorSubcoreMesh(core_axis_name="c", subcore_axis_name="s")
_sc_params = pltpu.CompilerParams(needs_layout_passes=False)


def _pre_body(conv_ref, lin_ref, linb_ref, a1_ref, a1b_ref, x_ref, o_ref):
    mrows = [
        jnp.dot(conv_ref[k], lin_ref[k], preferred_element_type=jnp.float32)
        for k in range(CH)
    ]
    w = jnp.concatenate(mrows + [a1_ref[:, :NFEAT], a1_ref[:, NFEAT:]], axis=0)
    acc = lax.dot_general(
        w, x_ref[...], (((1,), (1,)), ((), ())),
        preferred_element_type=jnp.float32,
    )
    bcs = [
        jnp.dot(conv_ref[k], linb_ref[...][k, :, None],
                preferred_element_type=jnp.float32)
        for k in range(CH)
    ]
    bvec = jnp.concatenate(
        bcs + [jnp.zeros((4, 1), jnp.float32), a1b_ref[...].reshape(4, 1)],
        axis=0,
    )
    o_ref[...] = acc + jnp.broadcast_to(bvec, acc.shape)


def _pre_call(conv_W, lin_W, lin_b, A1_W, A1_b, x):
    return pl.pallas_call(
        _pre_body,
        out_shape=jax.ShapeDtypeStruct((136, N), jnp.float32),
    )(conv_W, lin_W, lin_b, A1_W, A1_b, x)


@functools.partial(
    pl.kernel,
    out_type=jax.ShapeDtypeStruct((CH * E,), jnp.float32),
    mesh=_mesh,
    compiler_params=_sc_params,
    scratch_types=[
        pltpu.VMEM((8 * N,), jnp.float32),
        pltpu.VMEM((CH1,), jnp.int32),
        pltpu.VMEM((CH1,), jnp.int32),
        pltpu.VMEM((CH1,), jnp.int32),
        pltpu.VMEM((CH1,), jnp.int32),
        pltpu.VMEM((4 * CH1,), jnp.float32),
        pltpu.VMEM((4 * CH1,), jnp.float32),
        pltpu.VMEM((16,), jnp.float32),
        pltpu.VMEM((16,), jnp.float32),
        pltpu.SemaphoreType.DMA,
        pltpu.SemaphoreType.DMA,
        pltpu.SemaphoreType.DMA,
        pltpu.SemaphoreType.DMA,
    ],
)
def _sc_alpha(ot_hbm, ei_hbm, a2w_hbm, a2b_hbm, alT_hbm,
              uv_v, col0_v, col1_v, row0_v, row1_v, a0_v, a1_v, w_v, b_v,
              isem0, isem1, osem0, osem1):
    wid = lax.axis_index("s") * 2 + lax.axis_index("c")
    e0 = wid * EPT1
    nchunks = EPT1 // CH1
    cols_v = [col0_v, col1_v]
    rows_v = [row0_v, row1_v]
    as_v = [a0_v, a1_v]
    isems = [isem0, isem1]
    osems = [osem0, osem1]

    def fetch(ci, b):
        base = e0 + ci * CH1
        pltpu.async_copy(ei_hbm.at[pl.ds(E + base, CH1)], cols_v[b], isems[b])
        pltpu.async_copy(ei_hbm.at[pl.ds(base, CH1)], rows_v[b], isems[b])

    def drain_in(b):
        pltpu.make_async_copy(ei_hbm.at[pl.ds(0, CH1)], cols_v[b], isems[b]).wait()
        pltpu.make_async_copy(ei_hbm.at[pl.ds(0, CH1)], rows_v[b], isems[b]).wait()

    def store_out(ci, b):
        base = e0 + ci * CH1
        for c in range(4):
            pltpu.async_copy(
                as_v[b].at[pl.ds(c * CH1, CH1)],
                alT_hbm.at[pl.ds(c * E + base, CH1)],
                osems[b],
            )

    def drain_out(b):
        for c in range(4):
            pltpu.make_async_copy(
                as_v[b].at[pl.ds(c * CH1, CH1)],
                alT_hbm.at[pl.ds(0, CH1)],
                osems[b],
            ).wait()

    fetch(0, 0)
    pltpu.sync_copy(ot_hbm.at[pl.ds(UOFF, 8 * N)], uv_v)
    pltpu.sync_copy(a2w_hbm, w_v)
    pltpu.sync_copy(a2b_hbm, b_v)
    wvec = w_v[...]
    bvec = b_v[...]
    ws = [[wvec[c * 4 + j] for j in range(4)] for c in range(4)]
    bs = [bvec[c] for c in range(4)]

    for ci in range(nchunks):
        b = ci & 1
        if ci + 1 < nchunks:
            fetch(ci + 1, 1 - b)
        drain_in(b)
        if ci >= 2:
            drain_out(b)

        @functools.partial(plsc.parallel_loop, 0, CH1 // 16, unroll=2)
        def g_body(g):
            off = g * 16
            cols = cols_v[b][pl.ds(off, 16)]
            rows = rows_v[b][pl.ds(off, 16)] + (4 * N)
            t = [
                plsc.load_gather(uv_v, [cols + (c * N)])
                + plsc.load_gather(uv_v, [rows + (c * N)])
                for c in range(4)
            ]
            s = [
                bs[c]
                + t[0] * ws[c][0]
                + t[1] * ws[c][1]
                + t[2] * ws[c][2]
                + t[3] * ws[c][3]
                for c in range(4)
            ]
            m = jnp.maximum(jnp.maximum(s[0], s[1]), jnp.maximum(s[2], s[3]))
            ex = [jnp.exp(s[c] - m) for c in range(4)]
            r = 1.0 / ((ex[0] + ex[1]) + (ex[2] + ex[3]))
            for c in range(4):
                as_v[b][pl.ds(c * CH1 + off, 16)] = ex[c] * r

        store_out(ci, b)

    drain_out((nchunks - 2) & 1)
    drain_out((nchunks - 1) & 1)


@functools.partial(
    pl.kernel,
    out_type=jax.ShapeDtypeStruct((HIDDEN * N,), jnp.float32),
    mesh=_mesh,
    compiler_params=_sc_params,
    scratch_types=[
        pltpu.VMEM((FPT * N,), jnp.float32),
        pltpu.VMEM((FPT * N,), jnp.float32),
        pltpu.VMEM((CH2,), jnp.int32),
        pltpu.VMEM((CH2,), jnp.int32),
        pltpu.VMEM((CH2,), jnp.int32),
        pltpu.VMEM((CH2,), jnp.int32),
        pltpu.VMEM((CH2,), jnp.float32),
        pltpu.VMEM((CH2,), jnp.float32),
        pltpu.SemaphoreType.DMA,
        pltpu.SemaphoreType.DMA,
    ],
)
def _sc_agg(ot_hbm, ei_hbm, alT_hbm, pout_hbm,
            ct_v, out_v, col0_v, col1_v, row0_v, row1_v, al0_v, al1_v,
            sem0, sem1):
    wid = lax.axis_index("s") * 2 + lax.axis_index("c")
    ch = wid // (NTILES // CH)
    sems = [sem0, sem1]
    cols_v = [col0_v, col1_v]
    rows_v = [row0_v, row1_v]
    als_v = [al0_v, al1_v]
    nchunks = E // CH2

    def start_fetch(ci, b):
        base = ci * CH2
        pltpu.async_copy(ei_hbm.at[pl.ds(E + base, CH2)], cols_v[b], sems[b])
        pltpu.async_copy(ei_hbm.at[pl.ds(base, CH2)], rows_v[b], sems[b])
        pltpu.async_copy(alT_hbm.at[pl.ds(ch * E + base, CH2)], als_v[b], sems[b])

    def drain(b):
        pltpu.make_async_copy(ei_hbm.at[pl.ds(0, CH2)], cols_v[b], sems[b]).wait()
        pltpu.make_async_copy(ei_hbm.at[pl.ds(0, CH2)], rows_v[b], sems[b]).wait()
        pltpu.make_async_copy(alT_hbm.at[pl.ds(0, CH2)], als_v[b], sems[b]).wait()

    start_fetch(0, 0)
    pltpu.sync_copy(ot_hbm.at[pl.ds(wid * (FPT * N), FPT * N)], ct_v)

    def zero_body(i, _):
        out_v[pl.ds(i * 16, 16)] = jnp.zeros((16,), jnp.float32)
        return 0

    lax.fori_loop(0, FPT * N // 16, zero_body, 0, unroll=10)

    def process(b):
        def g_body(g, _):
            off = g * 16
            cols = cols_v[b][pl.ds(off, 16)]
            rows = rows_v[b][pl.ds(off, 16)]
            a = als_v[b][pl.ds(off, 16)]
            for f in range(FPT):
                vals = plsc.load_gather(ct_v, [cols + (f * N)]) * a
                plsc.addupdate_scatter(out_v, [rows + (f * N)], vals)
            return 0

        lax.fori_loop(0, CH2 // 16, g_body, 0)

    def pair_body(ci2, _):
        for b in range(2):
            ci = ci2 * 2 + b
            nxt = jnp.minimum(ci + 1, nchunks - 1)
            start_fetch(nxt, 1 - b)
            drain(b)
            process(b)
        return 0

    lax.fori_loop(0, nchunks // 2, pair_body, 0)
    drain(0)
    pltpu.sync_copy(out_v, pout_hbm.at[pl.ds(wid * (FPT * N), FPT * N)])


def _post_body(st_ref, cb_ref, wc_ref, bc_ref, h_ref, o_ref):
    y = st_ref[...] + jnp.broadcast_to(cb_ref[...], st_ref.shape)
    ri = lax.broadcasted_iota(jnp.int32, (HIDDEN, HIDDEN), 0) // PCD
    ci = lax.broadcasted_iota(jnp.int32, (HIDDEN, HIDDEN), 1) // PCD
    bm = jnp.where(ri == ci, 1.0, 0.0).astype(jnp.float32)
    nsq = jnp.dot(bm, y * y, preferred_element_type=jnp.float32)
    ht = y / jnp.maximum(jnp.sqrt(nsq), 1e-12)
    h = ht.T
    h_ref[...] = h
    o_ref[...] = (
        lax.dot_general(h, wc_ref[...], (((1,), (1,)), ((), ())),
                        preferred_element_type=jnp.float32)
        + jnp.broadcast_to(bc_ref[...], (N, 2))
    )


def _post_call(ST, ch_bias, cls_W, cls_b):
    return pl.pallas_call(
        _post_body,
        out_shape=[
            jax.ShapeDtypeStruct((N, 128), jnp.float32),
            jax.ShapeDtypeStruct((N, 2), jnp.float32),
        ],
    )(ST, ch_bias, cls_W, cls_b)


def kernel(x, edge_index, A1_W, A1_b, A2_W, A2_b, lin_W, lin_b, conv_W,
           ch_bias, cls_W, cls_b):
    ei_flat = edge_index.astype(jnp.int32).reshape(-1)

    OT = _pre_call(conv_W, lin_W, lin_b, A1_W, A1_b, x)
    ot_flat = OT.reshape(-1)

    a2w = A2_W.reshape(-1)
    a2b = jnp.concatenate([A2_b, jnp.zeros((12,), jnp.float32)])
    alT = _sc_alpha(ot_flat, ei_flat, a2w, a2b)

    pout = _sc_agg(ot_flat, ei_flat, alT)
    ST = pout.reshape(HIDDEN, N)

    h, out = _post_call(ST, ch_bias.reshape(HIDDEN, 1), cls_W, cls_b[None, :])
    return (h, out)
